# Initial kernel scaffold; baseline (speedup 1.0000x reference)
import functools
import math

import jax
import jax.numpy as jnp
from jax import lax
from jax.experimental import pallas as pl
from jax.experimental.pallas import tpu as pltpu
from jax.experimental.pallas import tpu_sc as plsc

B, S, M, D, L = 1024, 50, 24, 64, 4
P = B * S
NW = 32
PPW = P // NW
C = 16
NCHUNK = PPW // C
ROWS = C * M
NG = ROWS // 128
MPL = M // L


def _time_embed_body(td_ref, mask_ref, divf_ref, phase_ref, out_ref):
    td = td_ref[...] * mask_ref[...]
    row = lax.broadcasted_iota(jnp.int32, (S, S), 0)
    col = lax.broadcasted_iota(jnp.int32, (S, S), 1)
    tri = (row < col).astype(jnp.float32)
    t = jnp.dot(td, tri, preferred_element_type=jnp.float32)
    arg = t[:, :, None] * divf_ref[...][0][None, None, :] + phase_ref[...][0][None, None, :]
    out_ref[...] = jnp.sin(arg)


def _time_embed(time_delta, maskf, divf, phase):
    bb = 256
    return pl.pallas_call(
        _time_embed_body,
        grid=(B // bb,),
        in_specs=[
            pl.BlockSpec((bb, S), lambda i: (i, 0)),
            pl.BlockSpec((bb, S), lambda i: (i, 0)),
            pl.BlockSpec((1, D), lambda i: (0, 0)),
            pl.BlockSpec((1, D), lambda i: (0, 0)),
        ],
        out_specs=pl.BlockSpec((bb, S, D), lambda i: (i, 0, 0)),
        out_shape=jax.ShapeDtypeStruct((B, S, D), jnp.float32),
    )(time_delta, maskf, divf, phase)


def _sc_body(idx_hbm, te_hbm, table_hbm, out_hbm, idx_v, rows_v, te_v, out_v, sem):
    wid = lax.axis_index("s") * 2 + lax.axis_index("c")

    def chunk_body(c, carry):
        g = wid * NCHUNK + c
        base_pair = g * C
        pltpu.sync_copy(idx_hbm.at[pl.ds(g * NG, NG)], idx_v)
        pltpu.sync_copy(te_hbm.at[pl.ds(base_pair, C)], te_v)
        cps = [
            pltpu.async_copy(table_hbm.at[idx_v.at[j]],
                             rows_v.at[pl.ds(j * 128, 128)], sem)
            for j in range(NG)
        ]
        for cp in cps:
            cp.wait()

        def pair_body(p, carry2):
            for db in range(D // 16):
                sl = pl.ds(db * 16, 16)
                acc = te_v[p, sl]
                for lev in range(L):
                    for j in range(MPL):
                        acc = acc + rows_v[p * M + lev * MPL + j, sl]
                    out_v[p * L + lev, sl] = acc
            return carry2

        lax.fori_loop(0, C, pair_body, 0)
        pltpu.sync_copy(out_v, out_hbm.at[pl.ds(base_pair * L, C * L)])
        return carry

    lax.fori_loop(0, NCHUNK, chunk_body, 0)


@functools.partial(
    pl.kernel,
    out_type=jax.ShapeDtypeStruct((P * L, D), jnp.float32),
    mesh=plsc.VectorSubcoreMesh(core_axis_name="c", subcore_axis_name="s"),
    scratch_types=[
        pltpu.VMEM((NG, 128), jnp.int32),
        pltpu.VMEM((ROWS, D), jnp.float32),
        pltpu.VMEM((C, D), jnp.float32),
        pltpu.VMEM((C * L, D), jnp.float32),
        pltpu.SemaphoreType.DMA,
    ],
)
def _sc_gather(idx_hbm, te_hbm, table_hbm, out_hbm, idx_v, rows_v, te_v, out_v, sem):
    _sc_body(idx_hbm, te_hbm, table_hbm, out_hbm, idx_v, rows_v, te_v, out_v, sem)


def kernel(dynamic_indices, time_delta, event_mask, table, sin_div_term, cos_div_term):
    idx2d = dynamic_indices.astype(jnp.int32).reshape(P * M // 128, 128)
    maskf = event_mask.astype(jnp.float32)
    divf = jnp.stack([sin_div_term, cos_div_term], axis=-1).reshape(1, D)
    phase = jnp.tile(jnp.array([0.0, math.pi / 2], jnp.float32), D // 2).reshape(1, D)
    te = _time_embed(time_delta, maskf, divf, phase).reshape(P, D)
    out = _sc_gather(idx2d, te, table)
    return out.reshape(B, S, L, D)


# speedup vs baseline: 7.4601x; 7.4601x over previous
import functools
import math

import jax
import jax.numpy as jnp
from jax import lax
from jax.experimental import pallas as pl
from jax.experimental.pallas import tpu as pltpu
from jax.experimental.pallas import tpu_sc as plsc

B, S, M, D, L = 1024, 50, 24, 64, 4
P = B * S
NW = 32
PPW = P // NW
C = 16
NCHUNK = PPW // C
ROWS = C * M
NG = ROWS // 128
MPL = M // L


def _time_embed_body(td_ref, mask_ref, divf_ref, phase_ref, out_ref):
    td = td_ref[...] * mask_ref[...]
    row = lax.broadcasted_iota(jnp.int32, (S, S), 0)
    col = lax.broadcasted_iota(jnp.int32, (S, S), 1)
    tri = (row < col).astype(jnp.float32)
    t = jnp.dot(td, tri, preferred_element_type=jnp.float32,
                precision=lax.Precision.HIGHEST)
    arg = t[:, :, None] * divf_ref[...][0][None, None, :] + phase_ref[...][0][None, None, :]
    out_ref[...] = jnp.sin(arg)


def _time_embed(time_delta, maskf, divf, phase):
    bb = 256
    return pl.pallas_call(
        _time_embed_body,
        grid=(B // bb,),
        in_specs=[
            pl.BlockSpec((bb, S), lambda i: (i, 0)),
            pl.BlockSpec((bb, S), lambda i: (i, 0)),
            pl.BlockSpec((1, D), lambda i: (0, 0)),
            pl.BlockSpec((1, D), lambda i: (0, 0)),
        ],
        out_specs=pl.BlockSpec((bb, S, D), lambda i: (i, 0, 0)),
        out_shape=jax.ShapeDtypeStruct((B, S, D), jnp.float32),
    )(time_delta, maskf, divf, phase)


def _sc_body(idx_hbm, te_hbm, table_hbm, out_hbm, idx_v, rows_v, te_v, out_v, sem):
    wid = lax.axis_index("s") * 2 + lax.axis_index("c")

    def chunk_body(c, carry):
        g = wid * NCHUNK + c
        base_pair = g * C
        pltpu.sync_copy(idx_hbm.at[pl.ds(g * ROWS, ROWS)], idx_v)
        pltpu.sync_copy(te_hbm.at[pl.ds(base_pair, C)], te_v)
        cps = [
            pltpu.async_copy(table_hbm.at[idx_v.at[pl.ds(j * 128, 128)]],
                             rows_v.at[pl.ds(j * 128, 128)], sem)
            for j in range(NG)
        ]
        for cp in cps:
            cp.wait()

        def pair_body(p, carry2):
            for db in range(D // 16):
                sl = pl.ds(db * 16, 16)
                acc = te_v[p, sl]
                for lev in range(L):
                    for j in range(MPL):
                        acc = acc + rows_v[p * M + lev * MPL + j, sl]
                    out_v[p * L + lev, sl] = acc
            return carry2

        lax.fori_loop(0, C, pair_body, 0)
        pltpu.sync_copy(out_v, out_hbm.at[pl.ds(base_pair * L, C * L)])
        return carry

    lax.fori_loop(0, NCHUNK, chunk_body, 0)


@functools.partial(
    pl.kernel,
    out_type=jax.ShapeDtypeStruct((P * L, D), jnp.float32),
    mesh=plsc.VectorSubcoreMesh(core_axis_name="c", subcore_axis_name="s"),
    compiler_params=pltpu.CompilerParams(use_tc_tiling_on_sc=False),
    scratch_types=[
        pltpu.VMEM((ROWS,), jnp.int32),
        pltpu.VMEM((ROWS, D), jnp.float32),
        pltpu.VMEM((C, D), jnp.float32),
        pltpu.VMEM((C * L, D), jnp.float32),
        pltpu.SemaphoreType.DMA,
    ],
)
def _sc_gather(idx_hbm, te_hbm, table_hbm, out_hbm, idx_v, rows_v, te_v, out_v, sem):
    _sc_body(idx_hbm, te_hbm, table_hbm, out_hbm, idx_v, rows_v, te_v, out_v, sem)


def kernel(dynamic_indices, time_delta, event_mask, table, sin_div_term, cos_div_term):
    idx2d = dynamic_indices.astype(jnp.int32).reshape(P * M)
    maskf = event_mask.astype(jnp.float32)
    divf = jnp.stack([sin_div_term, cos_div_term], axis=-1).reshape(1, D)
    phase = jnp.tile(jnp.array([0.0, math.pi / 2], jnp.float32), D // 2).reshape(1, D)
    te = _time_embed(time_delta, maskf, divf, phase).reshape(P, D)
    out = _sc_gather(idx2d, te, table)
    return out.reshape(B, S, L, D)


# speedup vs baseline: 10.7414x; 1.4399x over previous
import functools
import math

import jax
import jax.numpy as jnp
from jax import lax
from jax.experimental import pallas as pl
from jax.experimental.pallas import tpu as pltpu
from jax.experimental.pallas import tpu_sc as plsc

B, S, M, D, L = 1024, 50, 24, 64, 4
P = B * S
NW = 32
PPW = P // NW
C = 16
NCHUNK = PPW // C
ROWS = C * M
NG = ROWS // 128
MPL = M // L


def _time_embed_body(td_ref, mask_ref, divf_ref, phase_ref, out_ref):
    td = td_ref[...] * mask_ref[...]
    row = lax.broadcasted_iota(jnp.int32, (S, S), 0)
    col = lax.broadcasted_iota(jnp.int32, (S, S), 1)
    tri = (row < col).astype(jnp.float32)
    t = jnp.dot(td, tri, preferred_element_type=jnp.float32,
                precision=lax.Precision.HIGHEST)
    arg = t[:, :, None] * divf_ref[...][0][None, None, :] + phase_ref[...][0][None, None, :]
    out_ref[...] = jnp.sin(arg)


def _time_embed(time_delta, maskf, divf, phase):
    bb = 256
    return pl.pallas_call(
        _time_embed_body,
        grid=(B // bb,),
        in_specs=[
            pl.BlockSpec((bb, S), lambda i: (i, 0)),
            pl.BlockSpec((bb, S), lambda i: (i, 0)),
            pl.BlockSpec((1, D), lambda i: (0, 0)),
            pl.BlockSpec((1, D), lambda i: (0, 0)),
        ],
        out_specs=pl.BlockSpec((bb, S, D), lambda i: (i, 0, 0)),
        out_shape=jax.ShapeDtypeStruct((B, S, D), jnp.float32),
    )(time_delta, maskf, divf, phase)


def _sc_body(idx_hbm, te_hbm, table_hbm, out_hbm, idx_all,
             rows0, rows1, te0, te1, out0, out1, sg0, sg1, so0, so1):
    wid = lax.axis_index("s") * 2 + lax.axis_index("c")
    base_pair_w = wid * PPW
    pltpu.sync_copy(idx_hbm.at[pl.ds(wid * PPW * M, PPW * M)], idx_all)

    rows = (rows0, rows1)
    te = (te0, te1)
    out = (out0, out1)
    sg = (sg0, sg1)
    so = (so0, so1)

    def stage(c, b):
        for j in range(NG):
            pltpu.async_copy(
                table_hbm.at[idx_all.at[pl.ds(c * ROWS + j * 128, 128)]],
                rows[b].at[pl.ds(j * 128, 128)], sg[b])
        pltpu.async_copy(te_hbm.at[pl.ds(base_pair_w + c * C, C)], te[b], sg[b])

    def wait_stage(c, b):
        for j in range(NG):
            pltpu.make_async_copy(
                table_hbm.at[idx_all.at[pl.ds(c * ROWS + j * 128, 128)]],
                rows[b].at[pl.ds(j * 128, 128)], sg[b]).wait()
        pltpu.make_async_copy(
            te_hbm.at[pl.ds(base_pair_w + c * C, C)], te[b], sg[b]).wait()

    def compute(c, b):
        rv, tv, ov = rows[b], te[b], out[b]

        @plsc.parallel_loop(0, C, unroll=2)
        def pair_body(p):
            for db in range(D // 16):
                sl = pl.ds(db * 16, 16)
                acc = tv[p, sl]
                for lev in range(L):
                    for j in range(MPL):
                        acc = acc + rv[p * M + lev * MPL + j, sl]
                    ov[p * L + lev, sl] = acc

        pltpu.async_copy(
            out[b], out_hbm.at[pl.ds((base_pair_w + c * C) * L, C * L)], so[b])

    def wait_out(c, b):
        pltpu.make_async_copy(
            out[b], out_hbm.at[pl.ds((base_pair_w + c * C) * L, C * L)],
            so[b]).wait()

    stage(0, 0)
    stage(1, 1)
    wait_stage(0, 0)
    compute(0, 0)
    stage(2, 0)
    wait_stage(1, 1)
    compute(1, 1)
    stage(3, 1)

    def loop_body(k, carry):
        c0 = 2 * k
        wait_out(c0 - 2, 0)
        wait_stage(c0, 0)
        compute(c0, 0)
        stage(c0 + 2, 0)
        wait_out(c0 - 1, 1)
        wait_stage(c0 + 1, 1)
        compute(c0 + 1, 1)
        stage(c0 + 3, 1)
        return carry

    lax.fori_loop(1, NCHUNK // 2 - 1, loop_body, 0)

    wait_out(NCHUNK - 4, 0)
    wait_stage(NCHUNK - 2, 0)
    compute(NCHUNK - 2, 0)
    wait_out(NCHUNK - 3, 1)
    wait_stage(NCHUNK - 1, 1)
    compute(NCHUNK - 1, 1)
    wait_out(NCHUNK - 2, 0)
    wait_out(NCHUNK - 1, 1)


@functools.partial(
    pl.kernel,
    out_type=jax.ShapeDtypeStruct((P * L, D), jnp.float32),
    mesh=plsc.VectorSubcoreMesh(core_axis_name="c", subcore_axis_name="s"),
    compiler_params=pltpu.CompilerParams(use_tc_tiling_on_sc=False),
    scratch_types=[
        pltpu.VMEM((PPW * M,), jnp.int32),
        pltpu.VMEM((ROWS, D), jnp.float32),
        pltpu.VMEM((ROWS, D), jnp.float32),
        pltpu.VMEM((C, D), jnp.float32),
        pltpu.VMEM((C, D), jnp.float32),
        pltpu.VMEM((C * L, D), jnp.float32),
        pltpu.VMEM((C * L, D), jnp.float32),
        pltpu.SemaphoreType.DMA,
        pltpu.SemaphoreType.DMA,
        pltpu.SemaphoreType.DMA,
        pltpu.SemaphoreType.DMA,
    ],
)
def _sc_gather(idx_hbm, te_hbm, table_hbm, out_hbm, idx_all,
               rows0, rows1, te0, te1, out0, out1, sg0, sg1, so0, so1):
    _sc_body(idx_hbm, te_hbm, table_hbm, out_hbm, idx_all,
             rows0, rows1, te0, te1, out0, out1, sg0, sg1, so0, so1)


def kernel(dynamic_indices, time_delta, event_mask, table, sin_div_term, cos_div_term):
    idx2d = dynamic_indices.astype(jnp.int32).reshape(P * M)
    maskf = event_mask.astype(jnp.float32)
    divf = jnp.stack([sin_div_term, cos_div_term], axis=-1).reshape(1, D)
    phase = jnp.tile(jnp.array([0.0, math.pi / 2], jnp.float32), D // 2).reshape(1, D)
    te = _time_embed(time_delta, maskf, divf, phase).reshape(P, D)
    out = _sc_gather(idx2d, te, table)
    return out.reshape(B, S, L, D)
